# baseline (speedup 1.0000x reference)
import functools

import jax
import jax.numpy as jnp
from jax import lax
from jax.experimental import pallas as pl
from jax.experimental.pallas import tpu as pltpu
from jax.experimental.pallas import tpu_sc as plsc

NC = 2
NS = 16
NW = NC * NS

CHUNK = 8
NBUF = 3


@functools.lru_cache(maxsize=None)
def _make_gather(B: int, V: int, D: int):
    assert B % (NW * CHUNK) == 0
    b_per_w = B // NW
    nchunks = b_per_w // CHUNK
    mesh = plsc.VectorSubcoreMesh(core_axis_name="c", subcore_axis_name="s")

    @functools.partial(
        pl.kernel,
        mesh=mesh,
        out_type=jax.ShapeDtypeStruct((B, D), jnp.float32),
        scratch_types=[
            pltpu.VMEM((b_per_w,), jnp.int32),
            pltpu.VMEM((NBUF, CHUNK, D), jnp.float32),
        ]
        + [pltpu.SemaphoreType.DMA] * (2 * NBUF),
    )
    def emb(idx_hbm, table_hbm, out_hbm, idx_v, bufs, *sems):
        gsem = sems[:NBUF]
        wsem = sems[NBUF:]
        wid = lax.axis_index("s") * NC + lax.axis_index("c")
        base = wid * b_per_w
        pltpu.sync_copy(idx_hbm.at[pl.ds(base, b_per_w)], idx_v)

        def start_gather(c, b):
            pltpu.async_copy(
                table_hbm.at[idx_v.at[pl.ds(c * CHUNK, CHUNK)]],
                bufs.at[b],
                gsem[b],
            )

        def wait_gather(c, b):
            pltpu.make_async_copy(
                table_hbm.at[idx_v.at[pl.ds(c * CHUNK, CHUNK)]],
                bufs.at[b],
                gsem[b],
            ).wait()

        def start_write(c, b):
            pltpu.async_copy(
                bufs.at[b],
                out_hbm.at[pl.ds(base + c * CHUNK, CHUNK)],
                wsem[b],
            )

        def wait_write(c, b):
            pltpu.make_async_copy(
                bufs.at[b],
                out_hbm.at[pl.ds(base + c * CHUNK, CHUNK)],
                wsem[b],
            ).wait()

        for b in range(NBUF):
            start_gather(b, b)

        def body(i, _):
            for b in range(NBUF):
                c = i * NBUF + b
                wait_gather(c, b)
                start_write(c, b)
                nxt = c + NBUF

                @pl.when(nxt < nchunks)
                def _():
                    wait_write(c, b)
                    start_gather(nxt, b)

            return 0

        main = NBUF * (nchunks // NBUF)
        lax.fori_loop(0, nchunks // NBUF, body, 0)

        for c in range(main, nchunks):
            wait_gather(c, c % NBUF)
            start_write(c, c % NBUF)

        for c in range(nchunks - NBUF, nchunks):
            wait_write(c, c % NBUF)

    return emb


def kernel(inputs, table):
    V, D = table.shape
    idx = inputs.reshape(-1).astype(jnp.int32)
    out = _make_gather(idx.shape[0], V, D)(idx, table)
    return out.reshape(inputs.shape + (D,))
